# Initial kernel scaffold; baseline (speedup 1.0000x reference)
import jax
import jax.numpy as jnp
from jax.experimental import pallas as pl


def kernel(x, running_mean, running_var, alpha):
    raise NotImplementedError("write your pallas kernel here")


# speedup vs baseline: 8.5683x; 8.5683x over previous
import jax
import jax.numpy as jnp
from jax.experimental import pallas as pl
from jax.experimental.pallas import tpu as pltpu

_EPS = 1e-12
_L = 200


def _body(a_ref, cp_ref, m0_ref, v0_ref, x_ref, o_ref):
    A = a_ref[...]
    cp = cp_ref[...]
    mc = m0_ref[...]
    vc = v0_ref[...]
    T = x_ref.shape[0]
    for i in range(T // _L):
        sl = pl.ds(i * _L, _L)
        xc = x_ref[sl, :]
        md = jnp.dot(A, xc.astype(jnp.bfloat16),
                     preferred_element_type=jnp.float32)
        m = md + cp * mc
        mc = m[_L - 1:_L, :]
        d = xc - m
        w = (d * d).astype(jnp.bfloat16)
        vd = jnp.dot(A, w, preferred_element_type=jnp.float32)
        v = vd + cp * vc
        vc = v[_L - 1:_L, :]
        o_ref[sl, :] = d / (v * 4.0 + _EPS)


def kernel(x, running_mean, running_var, alpha):
    B, T, F = x.shape
    a = alpha.reshape(()).astype(jnp.float32)
    c = 1.0 - a
    idx = jnp.arange(_L, dtype=jnp.float32)
    expo = idx[:, None] - idx[None, :]
    Amat = jnp.where(expo >= 0.0,
                     a * jnp.power(c, jnp.maximum(expo, 0.0)),
                     0.0).astype(jnp.bfloat16)
    cp = jnp.power(c, idx + 1.0)[:, None]

    x2 = x.reshape(B * T, F)
    m0 = running_mean.reshape(1, F).astype(jnp.float32)
    v0 = running_var.reshape(1, F).astype(jnp.float32)

    out = pl.pallas_call(
        _body,
        grid=(B,),
        in_specs=[
            pl.BlockSpec((_L, _L), lambda i: (0, 0)),
            pl.BlockSpec((_L, 1), lambda i: (0, 0)),
            pl.BlockSpec((1, F), lambda i: (0, 0)),
            pl.BlockSpec((1, F), lambda i: (0, 0)),
            pl.BlockSpec((T, F), lambda i: (i, 0)),
        ],
        out_specs=pl.BlockSpec((T, F), lambda i: (i, 0)),
        out_shape=jax.ShapeDtypeStruct((B * T, F), jnp.float32),
        compiler_params=pltpu.CompilerParams(
            dimension_semantics=("parallel",),
            vmem_limit_bytes=50 * 1024 * 1024,
        ),
        name="online_norm",
    )(Amat, cp, m0, v0, x2)
    return out.reshape(B, T, F)


# speedup vs baseline: 28.8645x; 3.3688x over previous
import jax
import jax.numpy as jnp
from jax.experimental import pallas as pl
from jax.experimental.pallas import tpu as pltpu

_EPS = 1e-12
_L = 200


def _body(a_ref, cp_ref, m0_ref, v0_ref, x_ref, o_ref):
    A = a_ref[...]
    cp = cp_ref[...]
    mc = m0_ref[...]
    vc = v0_ref[...]
    T = x_ref.shape[1]
    for i in range(T // _L):
        sl = pl.ds(i * _L, _L)
        xc = x_ref[0, sl, :]
        md = jnp.dot(A, xc.astype(jnp.bfloat16),
                     preferred_element_type=jnp.float32)
        m = md + cp * mc
        mc = m[_L - 1:_L, :]
        d = xc - m
        w = (d * d).astype(jnp.bfloat16)
        vd = jnp.dot(A, w, preferred_element_type=jnp.float32)
        v = vd + cp * vc
        vc = v[_L - 1:_L, :]
        o_ref[0, sl, :] = d / (v * 4.0 + _EPS)


def kernel(x, running_mean, running_var, alpha):
    B, T, F = x.shape
    a = alpha.reshape(()).astype(jnp.float32)
    c = 1.0 - a
    idx = jnp.arange(_L, dtype=jnp.float32)
    expo = idx[:, None] - idx[None, :]
    Amat = jnp.where(expo >= 0.0,
                     a * jnp.power(c, jnp.maximum(expo, 0.0)),
                     0.0).astype(jnp.bfloat16)
    cp = jnp.power(c, idx + 1.0)[:, None]

    m0 = running_mean.reshape(1, F).astype(jnp.float32)
    v0 = running_var.reshape(1, F).astype(jnp.float32)

    out = pl.pallas_call(
        _body,
        grid=(B,),
        in_specs=[
            pl.BlockSpec((_L, _L), lambda i: (0, 0)),
            pl.BlockSpec((_L, 1), lambda i: (0, 0)),
            pl.BlockSpec((1, F), lambda i: (0, 0)),
            pl.BlockSpec((1, F), lambda i: (0, 0)),
            pl.BlockSpec((1, T, F), lambda i: (i, 0, 0)),
        ],
        out_specs=pl.BlockSpec((1, T, F), lambda i: (i, 0, 0)),
        out_shape=jax.ShapeDtypeStruct((B, T, F), jnp.float32),
        compiler_params=pltpu.CompilerParams(
            dimension_semantics=("parallel",),
            vmem_limit_bytes=50 * 1024 * 1024,
        ),
        name="online_norm",
    )(Amat, cp, m0, v0, x)
    return out
